# baseline (speedup 1.0000x reference)
import functools

import jax
import jax.numpy as jnp
from jax import lax
from jax.experimental import pallas as pl
from jax.experimental.pallas import tpu as pltpu
from jax.experimental.pallas import tpu_sc as plsc

GAMMA = 12.0
B = 16384
DIM = 128
NC = 2
NS = 16
L = 16
NW = NC * NS
BPW = B // NW
C = 64
NCH = BPW // C
G = DIM // L


def _sc_score(heads, rels, tails, entity, relation):
    mesh = plsc.VectorSubcoreMesh(core_axis_name="c", subcore_axis_name="s")

    @functools.partial(
        pl.kernel,
        mesh=mesh,
        compiler_params=pltpu.CompilerParams(needs_layout_passes=False),
        out_type=jax.ShapeDtypeStruct((B,), jnp.float32),
        scratch_types=[
            pltpu.VMEM((BPW,), jnp.int32),
            pltpu.VMEM((BPW,), jnp.int32),
            pltpu.VMEM((BPW,), jnp.int32),
            pltpu.VMEM((2, C, DIM), jnp.float32),
            pltpu.VMEM((2, C, DIM), jnp.float32),
            pltpu.VMEM((2, C, DIM), jnp.float32),
            pltpu.VMEM((L * L,), jnp.float32),
            pltpu.VMEM((BPW,), jnp.float32),
            pltpu.SemaphoreType.DMA,
            pltpu.SemaphoreType.DMA,
            pltpu.SemaphoreType.DMA,
        ],
    )
    def body(heads_hbm, rels_hbm, tails_hbm, ent_hbm, rel_hbm, out_hbm,
             h_idx, r_idx, t_idx, h_buf, r_buf, t_buf, tr, scores,
             sem_a, sem_b, sem_i):
        wid = lax.axis_index("s") * NC + lax.axis_index("c")
        base = wid * BPW
        sems = (sem_a, sem_b)
        lane = lax.iota(jnp.int32, L)

        ci = pltpu.async_copy(heads_hbm.at[pl.ds(base, BPW)], h_idx, sem_i)
        cr = pltpu.async_copy(rels_hbm.at[pl.ds(base, BPW)], r_idx, sem_i)
        ct = pltpu.async_copy(tails_hbm.at[pl.ds(base, BPW)], t_idx, sem_i)
        ci.wait()
        cr.wait()
        ct.wait()

        def fire(c, slot):
            sl = pl.ds(c * C, C)
            return (
                pltpu.async_copy(ent_hbm.at[h_idx.at[sl]], h_buf.at[slot], sems[slot]),
                pltpu.async_copy(rel_hbm.at[r_idx.at[sl]], r_buf.at[slot], sems[slot]),
                pltpu.async_copy(ent_hbm.at[t_idx.at[sl]], t_buf.at[slot], sems[slot]),
            )

        def wait(cps):
            for cp in cps:
                cp.wait()

        def compute(c, slot):
            def grp_body(g2, carry):
                r0 = g2 * L
                for j in range(L):
                    acc = jnp.zeros((L,), jnp.float32)
                    for g in range(G):
                        h = h_buf[slot, r0 + j, pl.ds(g * L, L)]
                        r = r_buf[slot, r0 + j, pl.ds(g * L, L)]
                        t = t_buf[slot, r0 + j, pl.ds(g * L, L)]
                        acc = acc + jnp.abs(h + r - t)
                    tr[pl.ds(j * L, L)] = acc
                sv = jnp.zeros((L,), jnp.float32)
                for i in range(L):
                    sv = sv + plsc.load_gather(tr, [lane * L + i])
                scores[pl.ds(c * C + r0, L)] = GAMMA - sv
                return carry

            lax.fori_loop(0, C // L, grp_body, 0)

        fire0 = fire(0, 0)

        def pair_body(c2, carry):
            c0 = 2 * c2
            cps1 = fire(c0 + 1, 1)
            wait(fire0)
            compute(c0, 0)

            @pl.when(c0 + 2 < NCH)
            def _():
                fire(c0 + 2, 0)

            wait(cps1)
            compute(c0 + 1, 1)
            return carry

        lax.fori_loop(0, NCH // 2, pair_body, 0)

        pltpu.sync_copy(scores, out_hbm.at[pl.ds(base, BPW)])

    return body(heads, rels, tails, entity, relation)


def kernel(sample, entity_embedding, relation_embedding):
    heads = sample[:, 0]
    rels = sample[:, 1]
    tails = sample[:, 2]
    return _sc_score(heads, rels, tails, entity_embedding, relation_embedding)


# speedup vs baseline: 1.0425x; 1.0425x over previous
import functools

import jax
import jax.numpy as jnp
from jax import lax
from jax.experimental import pallas as pl
from jax.experimental.pallas import tpu as pltpu
from jax.experimental.pallas import tpu_sc as plsc

GAMMA = 12.0
B = 16384
DIM = 128
NC = 2
NS = 16
L = 16
NW = NC * NS
BPW = B // NW
C = 64
NCH = BPW // C
G = DIM // L


def _sc_score(heads, rels, tails, entity, relation):
    mesh = plsc.VectorSubcoreMesh(core_axis_name="c", subcore_axis_name="s")

    @functools.partial(
        pl.kernel,
        mesh=mesh,
        compiler_params=pltpu.CompilerParams(needs_layout_passes=False),
        out_type=jax.ShapeDtypeStruct((B,), jnp.float32),
        scratch_types=[
            pltpu.VMEM((BPW,), jnp.int32),
            pltpu.VMEM((BPW,), jnp.int32),
            pltpu.VMEM((BPW,), jnp.int32),
            pltpu.VMEM((2, C, DIM), jnp.float32),
            pltpu.VMEM((2, C, DIM), jnp.float32),
            pltpu.VMEM((2, C, DIM), jnp.float32),
            pltpu.VMEM((L * L,), jnp.float32),
            pltpu.VMEM((BPW,), jnp.float32),
            pltpu.SemaphoreType.DMA((2,)),
            pltpu.SemaphoreType.DMA,
        ],
    )
    def body(heads_hbm, rels_hbm, tails_hbm, ent_hbm, rel_hbm, out_hbm,
             h_idx, r_idx, t_idx, h_buf, r_buf, t_buf, tr, scores,
             sem, sem_i):
        wid = lax.axis_index("s") * NC + lax.axis_index("c")
        base = wid * BPW
        lane = lax.iota(jnp.int32, L)

        ci = pltpu.async_copy(heads_hbm.at[pl.ds(base, BPW)], h_idx, sem_i)
        cr = pltpu.async_copy(rels_hbm.at[pl.ds(base, BPW)], r_idx, sem_i)
        ct = pltpu.async_copy(tails_hbm.at[pl.ds(base, BPW)], t_idx, sem_i)
        ci.wait()
        cr.wait()
        ct.wait()

        def fire(c, slot):
            sl = pl.ds(c * C, C)
            pltpu.async_copy(ent_hbm.at[h_idx.at[sl]], h_buf.at[slot], sem.at[slot])
            pltpu.async_copy(rel_hbm.at[r_idx.at[sl]], r_buf.at[slot], sem.at[slot])
            pltpu.async_copy(ent_hbm.at[t_idx.at[sl]], t_buf.at[slot], sem.at[slot])

        def wait_slot(slot):
            sl = pl.ds(0, C)
            for buf in (h_buf, r_buf, t_buf):
                pltpu.make_async_copy(
                    ent_hbm.at[h_idx.at[sl]], buf.at[slot], sem.at[slot]
                ).wait()

        fire(0, 0)

        def chunk_body(c, carry):
            slot = lax.rem(c, 2)

            @pl.when(c + 1 < NCH)
            def _():
                fire(c + 1, lax.rem(c + 1, 2))

            wait_slot(slot)

            def grp_body(g2, carry2):
                r0 = g2 * L
                for j in range(L):
                    acc = jnp.zeros((L,), jnp.float32)
                    for g in range(G):
                        h = h_buf[slot, r0 + j, pl.ds(g * L, L)]
                        r = r_buf[slot, r0 + j, pl.ds(g * L, L)]
                        t = t_buf[slot, r0 + j, pl.ds(g * L, L)]
                        acc = acc + jnp.abs(h + r - t)
                    tr[pl.ds(j * L, L)] = acc
                sv = jnp.zeros((L,), jnp.float32)
                for i in range(L):
                    sv = sv + plsc.load_gather(tr, [lane * L + i])
                scores[pl.ds(c * C + r0, L)] = GAMMA - sv
                return carry2

            lax.fori_loop(0, C // L, grp_body, 0)
            return carry

        lax.fori_loop(0, NCH, chunk_body, 0)

        pltpu.sync_copy(scores, out_hbm.at[pl.ds(base, BPW)])

    return body(heads, rels, tails, entity, relation)


def kernel(sample, entity_embedding, relation_embedding):
    heads = sample[:, 0]
    rels = sample[:, 1]
    tails = sample[:, 2]
    scores = _sc_score(heads, rels, tails, entity_embedding, relation_embedding)
    return scores.reshape(B, 1)


# speedup vs baseline: 1.0543x; 1.0112x over previous
import functools

import jax
import jax.numpy as jnp
from jax import lax
from jax.experimental import pallas as pl
from jax.experimental.pallas import tpu as pltpu
from jax.experimental.pallas import tpu_sc as plsc

GAMMA = 12.0
B = 16384
DIM = 128
NC = 2
NS = 16
L = 16
NW = NC * NS
BPW = B // NW
C = 64
NCH = BPW // C
G = DIM // L


def _sc_score(heads, rels, tails, entity, relation):
    mesh = plsc.VectorSubcoreMesh(core_axis_name="c", subcore_axis_name="s")

    @functools.partial(
        pl.kernel,
        mesh=mesh,
        compiler_params=pltpu.CompilerParams(needs_layout_passes=False),
        out_type=jax.ShapeDtypeStruct((B,), jnp.float32),
        scratch_types=[
            pltpu.VMEM((BPW,), jnp.int32),
            pltpu.VMEM((BPW,), jnp.int32),
            pltpu.VMEM((BPW,), jnp.int32),
            pltpu.VMEM((2, C, DIM), jnp.float32),
            pltpu.VMEM((2, C, DIM), jnp.float32),
            pltpu.VMEM((2, C, DIM), jnp.float32),
            pltpu.VMEM((L * L,), jnp.float32),
            pltpu.VMEM((BPW,), jnp.float32),
            pltpu.SemaphoreType.DMA((2,)),
            pltpu.SemaphoreType.DMA,
        ],
    )
    def body(heads_hbm, rels_hbm, tails_hbm, ent_hbm, rel_hbm, out_hbm,
             h_idx, r_idx, t_idx, h_buf, r_buf, t_buf, tr, scores,
             sem, sem_i):
        wid = lax.axis_index("s") * NC + lax.axis_index("c")
        base = wid * BPW
        lane = lax.iota(jnp.int32, L)

        ci = pltpu.async_copy(heads_hbm.at[pl.ds(base, BPW)], h_idx, sem_i)
        cr = pltpu.async_copy(rels_hbm.at[pl.ds(base, BPW)], r_idx, sem_i)
        ct = pltpu.async_copy(tails_hbm.at[pl.ds(base, BPW)], t_idx, sem_i)
        ci.wait()
        cr.wait()
        ct.wait()

        def fire(c, slot):
            sl = pl.ds(c * C, C)
            pltpu.async_copy(ent_hbm.at[h_idx.at[sl]], h_buf.at[slot], sem.at[slot])
            pltpu.async_copy(rel_hbm.at[r_idx.at[sl]], r_buf.at[slot], sem.at[slot])
            pltpu.async_copy(ent_hbm.at[t_idx.at[sl]], t_buf.at[slot], sem.at[slot])

        def wait_slot(slot):
            sl = pl.ds(0, C)
            for buf in (h_buf, r_buf, t_buf):
                pltpu.make_async_copy(
                    ent_hbm.at[h_idx.at[sl]], buf.at[slot], sem.at[slot]
                ).wait()

        fire(0, 0)

        def chunk_body(c, carry):
            slot = lax.rem(c, 2)

            @pl.when(c + 1 < NCH)
            def _():
                fire(c + 1, lax.rem(c + 1, 2))

            wait_slot(slot)

            def grp_body(g2, carry2):
                r0 = g2 * L

                def row_body(j, carry3):
                    acc = jnp.zeros((L,), jnp.float32)
                    for g in range(G):
                        h = h_buf[slot, r0 + j, pl.ds(g * L, L)]
                        r = r_buf[slot, r0 + j, pl.ds(g * L, L)]
                        t = t_buf[slot, r0 + j, pl.ds(g * L, L)]
                        acc = acc + jnp.abs(h + r - t)
                    tr[pl.ds(j * L, L)] = acc
                    return carry3

                lax.fori_loop(0, L, row_body, 0)
                sv = jnp.zeros((L,), jnp.float32)
                for i in range(L):
                    sv = sv + plsc.load_gather(tr, [lane * L + i])
                scores[pl.ds(c * C + r0, L)] = GAMMA - sv
                return carry2

            lax.fori_loop(0, C // L, grp_body, 0)
            return carry

        lax.fori_loop(0, NCH, chunk_body, 0)

        pltpu.sync_copy(scores, out_hbm.at[pl.ds(base, BPW)])

    return body(heads, rels, tails, entity, relation)


def kernel(sample, entity_embedding, relation_embedding):
    heads = sample[:, 0]
    rels = sample[:, 1]
    tails = sample[:, 2]
    scores = _sc_score(heads, rels, tails, entity_embedding, relation_embedding)
    return scores.reshape(B, 1)
